# baseline (speedup 1.0000x reference)
import jax
import jax.numpy as jnp
from jax import lax
from jax.experimental import pallas as pl
from jax.experimental.pallas import tpu as pltpu
from jax.experimental.pallas import tpu_sc as plsc

N = 10000
E = 320000
G = 128
D = 128
H = 32
C = 10
EPS = 1e-5

NW = 32
NP = 10112
RPT = NP // 16
EPT = E // NW
CW = 128
NG = 80
EPT_PAD = NG * CW


def _relu(t):
    return jnp.maximum(t, 0.0)


def _dot(a, b):
    return jnp.dot(a, b, preferred_element_type=jnp.float32)


def _make_sc_scatter(ncols, with_ew):
    mesh = plsc.VectorSubcoreMesh(core_axis_name="c", subcore_axis_name="s")
    out_type = [jax.ShapeDtypeStruct((2, NP, ncols), jnp.float32)]
    if with_ew:
        out_type.append(jax.ShapeDtypeStruct((NW, NG, CW), jnp.float32))
    scratch = [
        pltpu.VMEM((NG, CW), jnp.int32),
        pltpu.VMEM((NG, CW), jnp.int32),
        pltpu.VMEM((CW, ncols), jnp.float32),
        pltpu.VMEM((CW, ncols), jnp.float32),
        pltpu.SemaphoreType.DMA,
        pltpu.SemaphoreType.DMA,
        pltpu.SemaphoreType.DMA,
        pltpu.SemaphoreType.DMA,
        pltpu.VMEM_SHARED((NP, ncols), jnp.float32),
    ]
    if with_ew:
        scratch += [
            pltpu.VMEM((NP,), jnp.float32),
            pltpu.VMEM((NG, CW), jnp.float32),
        ]

    def body(v_hbm, srcs_hbm, dsts_hbm, zeros_hbm, *rest):
        if with_ew:
            (a0_hbm, out_hbm, ew_hbm, src_v, dst_v, rows0, rows1, sem0, sem1,
             sem2, sem3, acc, a0_v, ew_v) = rest
        else:
            (out_hbm, src_v, dst_v, rows0, rows1, sem0, sem1, sem2, sem3,
             acc) = rest
        c = lax.axis_index("c")
        s = lax.axis_index("s")
        b = c * 16 + s
        pltpu.sync_copy(srcs_hbm.at[b], src_v)
        pltpu.sync_copy(dsts_hbm.at[b], dst_v)
        pltpu.sync_copy(zeros_hbm.at[pl.ds(s * RPT, RPT)],
                        acc.at[pl.ds(s * RPT, RPT)])
        plsc.subcore_barrier()

        def step(jj, carry):
            j = jj * 2
            d0 = pltpu.async_copy(v_hbm.at[src_v.at[j]], rows0, sem0)
            d1 = pltpu.async_copy(v_hbm.at[src_v.at[j + 1]], rows1, sem1)
            d0.wait()
            s0 = pltpu.async_copy(rows0, acc.at[dst_v.at[j]], sem2, add=True)
            d1.wait()
            s1 = pltpu.async_copy(rows1, acc.at[dst_v.at[j + 1]], sem3,
                                  add=True)
            s0.wait()
            s1.wait()
            return carry

        lax.fori_loop(0, NG // 2, step, 0)

        if with_ew:
            pltpu.sync_copy(a0_hbm, a0_v)

            def ewstep(j, carry):
                for k in range(CW // 16):
                    si = src_v[j, pl.ds(k * 16, 16)]
                    di = dst_v[j, pl.ds(k * 16, 16)]
                    av = plsc.load_gather(a0_v, [si])
                    bv = plsc.load_gather(a0_v, [di])
                    ew_v[j, pl.ds(k * 16, 16)] = av * bv
                return carry

            lax.fori_loop(0, NG, ewstep, 0)
            pltpu.sync_copy(ew_v, ew_hbm.at[b])

        plsc.subcore_barrier()
        pltpu.sync_copy(acc.at[pl.ds(s * RPT, RPT)],
                        out_hbm.at[c, pl.ds(s * RPT, RPT)])

    return pl.kernel(body,
                     out_type=tuple(out_type) if with_ew else out_type[0],
                     mesh=mesh, scratch_types=scratch,
                     compiler_params=pltpu.CompilerParams(
                         use_tc_tiling_on_sc=False,
                         needs_layout_passes=False))


def _pre_body(x_ref, w_ref, o_ref):
    o_ref[...] = _dot(x_ref[...], w_ref[...])


def _gin_body(p0, p1, z, w1, b1, w2, b2, o_ref):
    h = p0[...] + p1[...] + z[...]
    h1 = _relu(_dot(h, w1[...]) + b1[...])
    o_ref[...] = _relu(_dot(h1, w2[...]) + b2[...])


def _gin0_body(pa0, pa1, pb0, pb1, z, w1, b1, w2, b2, o_ref):
    agg = jnp.concatenate([pa0[...] + pa1[...], pb0[...] + pb1[...]], 1)
    h = agg + z[...]
    h1 = _relu(_dot(h, w1[...]) + b1[...])
    o_ref[...] = _relu(_dot(h1, w2[...]) + b2[...])


def _big_body(p0, p1, z, ucat, w1, b1, w2, b2, wi1, bi1, wi2, bi2, wi3, bi3,
              z1_o, as_o, vc_o):
    h = p0[...] + p1[...] + z[...]
    h1 = _relu(_dot(h, w1[...]) + b1[...])
    z1 = _relu(_dot(h1, w2[...]) + b2[...])
    z1_o[...] = z1
    mask = lax.broadcasted_iota(jnp.int32, (NP, 1), 0) < N

    def inorm(t):
        m = jnp.sum(jnp.where(mask, t, 0.0), 0, keepdims=True) / N
        d = t - m
        v = jnp.sum(jnp.where(mask, d * d, 0.0), 0, keepdims=True) / N
        return d / jnp.sqrt(v + EPS)

    h = _relu(inorm(_dot(z1, wi1[...]) + bi1[...]))
    h = _relu(inorm(_dot(h, wi2[...]) + bi2[...]))
    ib = _dot(h, wi3[...]) + bi3[...]
    e = jnp.exp(ib - jnp.max(ib, 1, keepdims=True))
    a = e / jnp.sum(e, 1, keepdims=True)
    as_o[...] = a
    u2 = ucat[...]
    vc_o[...] = jnp.concatenate(
        [u2[:, :H] * a[:, 0:1], u2[:, H:] * a[:, 1:2]], 1)


def _mid2_body(p0, p1, u, a, b1c, b1o, w2c, b2c, w2o, b2o, w1c, w1o,
               u_o, v_o):
    a0 = a[:, 0:1]
    a1 = a[:, 1:2]
    s = p0[...] + p1[...]
    hc = _relu(s[:, :H] * a0 + u[:, :H] + b1c[...])
    ho = _relu(s[:, H:] * a1 + u[:, H:] + b1o[...])
    zc = _relu(_dot(hc, w2c[...]) + b2c[...])
    zo = _relu(_dot(ho, w2o[...]) + b2o[...])
    uc = _dot(zc, w1c[...])
    uo = _dot(zo, w1o[...])
    u_o[...] = jnp.concatenate([uc, uo], 1)
    v_o[...] = jnp.concatenate([uc * a0, uo * a1], 1)


def _fin_body(p0, p1, u, a, b1c, b1o, w2c, b2c, w2o, b2o, z1, bb, perm,
              w11, b11, w12, b12, wo1, bo1, wo2, bo2, wc1, bc1, wc2, bc2,
              wco1, bco1, wco2, bco2, ws1, bs1, ws2, bs2,
              h1_o, g1_o, hm_o, gm_o, hco_o, hres_o, hs_o):
    a0 = a[:, 0:1]
    a1 = a[:, 1:2]
    s = p0[...] + p1[...]
    hc = _relu(s[:, :H] * a0 + u[:, :H] + b1c[...])
    ho = _relu(s[:, H:] * a1 + u[:, H:] + b1o[...])
    z_m = _relu(_dot(hc, w2c[...]) + b2c[...])
    z_r = _relu(_dot(ho, w2o[...]) + b2o[...])
    oh = (bb[...] == lax.broadcasted_iota(jnp.int32, (G, 1), 0)
          ).astype(jnp.float32)
    g1 = _dot(oh, z1[...])
    gm = _dot(oh, z_m)
    gr = _dot(oh, z_r)
    g1_o[...] = g1
    gm_o[...] = gm
    grp = _dot(perm[...], gr)
    gmp = _dot(perm[...], gm)

    def fc(t, w1_, b1_, w2_, b2_):
        return _dot(_relu(_dot(t, w1_[...]) + b1_[...]), w2_[...]) + b2_[...]

    h1_o[...] = fc(g1, w11, b11, w12, b12)
    hm_o[...] = fc(gm, wo1, bo1, wo2, bo2)
    hco_o[...] = fc(jnp.concatenate([grp, gm], 1), wco1, bco1, wco2, bco2)
    hres_o[...] = fc(gr, wc1, bc1, wc2, bc2)
    hs_o[...] = fc(jnp.concatenate([gr, gmp], 1), ws1, bs1, ws2, bs2)


def _tc(body, out_shape):
    return pl.pallas_call(body, out_shape=out_shape)


def kernel(x, edge_index, batch, y, params):
    f32 = jnp.float32
    sds = jax.ShapeDtypeStruct

    x_pad = jnp.pad(x, ((0, NP - N), (0, 0)))

    def pad_idx(v):
        vb = v.reshape(NW, EPT)
        vb = jnp.pad(vb, ((0, 0), (0, EPT_PAD - EPT)), constant_values=N)
        return vb.reshape(NW, NG, CW)

    srcs = pad_idx(edge_index[0])
    dsts = pad_idx(edge_index[1])
    bb = jnp.pad(batch, (0, NP - N), constant_values=G).reshape(1, NP)
    zeros32 = jnp.zeros((NP, H), f32)
    zeros64 = jnp.zeros((NP, 2 * H), f32)
    ridx = jax.random.permutation(jax.random.key(42), G)
    perm = jnp.zeros((G, G), f32).at[jnp.arange(G), ridx].set(1.0)

    def r2(b):
        return b.reshape(1, -1)

    sc32 = _make_sc_scatter(H, False)
    sc64ew = _make_sc_scatter(2 * H, True)
    sc64 = _make_sc_scatter(2 * H, False)

    gp = params["gcn1"]
    cp = params["context"]
    op = params["objects"]

    w1cat0 = jnp.concatenate([cp[0][0], op[0][0]], 1)
    ucat = _tc(_pre_body, sds((NP, 2 * H), f32))(x_pad, w1cat0)

    pa = sc64(x_pad[:, :2 * H], srcs, dsts, zeros64)
    pb = sc64(x_pad[:, 2 * H:], srcs, dsts, zeros64)
    z = _tc(_gin0_body, sds((NP, H), f32))(
        pa[0], pa[1], pb[0], pb[1], x_pad,
        gp[0][0], r2(gp[0][1]), gp[0][2], r2(gp[0][3]))
    p = sc32(z, srcs, dsts, zeros32)
    z = _tc(_gin_body, sds((NP, H), f32))(
        p[0], p[1], z, gp[1][0], r2(gp[1][1]), gp[1][2], r2(gp[1][3]))
    p = sc32(z, srcs, dsts, zeros32)

    wib = params["mlp_IB"]
    z1, assign, vcat = _tc(
        _big_body,
        (sds((NP, H), f32), sds((NP, 2), f32), sds((NP, 2 * H), f32)))(
        p[0], p[1], z, ucat, gp[2][0], r2(gp[2][1]), gp[2][2], r2(gp[2][3]),
        wib[0], r2(wib[1]), wib[2], r2(wib[3]), wib[4], r2(wib[5]))

    a0flat = assign[:, 0]
    p, ew = sc64ew(vcat, srcs, dsts, zeros64, a0flat)
    for l in range(2):
        args = (p[0], p[1], ucat, assign,
                r2(cp[l][1]), r2(op[l][1]), cp[l][2], r2(cp[l][3]),
                op[l][2], r2(op[l][3]), cp[l + 1][0], op[l + 1][0])
        ucat, vcat = _tc(
            _mid2_body, (sds((NP, 2 * H), f32), sds((NP, 2 * H), f32)))(*args)
        p = sc64(vcat, srcs, dsts, zeros64)

    m1 = params["mlp1"]
    mo = params["mlp_o"]
    mc = params["mlp_c"]
    mco = params["mlp_co"]
    ms = params["mlp_co_s"]
    h1, g1, hm, gm, hco, hres, hs = _tc(
        _fin_body,
        (sds((G, C), f32), sds((G, H), f32), sds((G, C), f32),
         sds((G, H), f32), sds((G, C), f32), sds((G, C), f32),
         sds((G, C), f32)))(
        p[0], p[1], ucat, assign,
        r2(cp[2][1]), r2(op[2][1]), cp[2][2], r2(cp[2][3]),
        op[2][2], r2(op[2][3]), z1, bb, perm,
        m1[0], r2(m1[1]), m1[2], r2(m1[3]),
        mo[0], r2(mo[1]), mo[2], r2(mo[3]),
        mc[0], r2(mc[1]), mc[2], r2(mc[3]),
        mco[0], r2(mco[1]), mco[2], r2(mco[3]),
        ms[0], r2(ms[1]), ms[2], r2(ms[3]))

    assignment = assign[:N]
    a0_out = assignment[:, 0]
    ew_o = ew.reshape(NW, EPT_PAD)[:, :EPT].reshape(E, 1)
    y_shuf = y[ridx]
    return (h1, g1, hm, gm, assignment, hco, hres, ew_o, a0_out, hs, y_shuf)


# speedup vs baseline: 1.9797x; 1.9797x over previous
import jax
import jax.numpy as jnp
from jax import lax
from jax.experimental import pallas as pl
from jax.experimental.pallas import tpu as pltpu
from jax.experimental.pallas import tpu_sc as plsc

N = 10000
E = 320000
G = 128
D = 128
H = 32
C = 10
EPS = 1e-5

NW = 32
NP = 10112
RPT = NP // 16
EPT = E // NW
CW = 128
NG = 79
EPT_PAD = NG * CW


def _relu(t):
    return jnp.maximum(t, 0.0)


def _dot(a, b):
    return jnp.dot(a, b, preferred_element_type=jnp.float32)


def _make_sc_scatter(ncols, with_ew):
    mesh = plsc.VectorSubcoreMesh(core_axis_name="c", subcore_axis_name="s")
    out_type = [jax.ShapeDtypeStruct((2, NP, ncols), jnp.float32)]
    if with_ew:
        out_type.append(jax.ShapeDtypeStruct((NW, NG, CW), jnp.float32))
    scratch = [
        pltpu.VMEM((NG, CW), jnp.int32),
        pltpu.VMEM((NG, CW), jnp.int32),
        pltpu.VMEM((CW, ncols), jnp.float32),
        pltpu.VMEM((CW, ncols), jnp.float32),
        pltpu.SemaphoreType.DMA,
        pltpu.SemaphoreType.DMA,
        pltpu.SemaphoreType.DMA,
        pltpu.SemaphoreType.DMA,
        pltpu.VMEM_SHARED((NP, ncols), jnp.float32),
    ]
    if with_ew:
        scratch += [
            pltpu.VMEM((NP,), jnp.float32),
            pltpu.VMEM((NG, CW), jnp.float32),
        ]

    def body(v_hbm, srcs_hbm, dsts_hbm, zeros_hbm, *rest):
        if with_ew:
            (a0_hbm, out_hbm, ew_hbm, src_v, dst_v, rows0, rows1, sem0, sem1,
             sem2, sem3, acc, a0_v, ew_v) = rest
        else:
            (out_hbm, src_v, dst_v, rows0, rows1, sem0, sem1, sem2, sem3,
             acc) = rest
        c = lax.axis_index("c")
        s = lax.axis_index("s")
        b = c * 16 + s
        pltpu.sync_copy(srcs_hbm.at[b], src_v)
        pltpu.sync_copy(dsts_hbm.at[b], dst_v)
        pltpu.sync_copy(zeros_hbm.at[pl.ds(s * RPT, RPT)],
                        acc.at[pl.ds(s * RPT, RPT)])
        plsc.subcore_barrier()

        def step(jj, carry):
            j = jj * 2
            d0 = pltpu.async_copy(v_hbm.at[src_v.at[j]], rows0, sem0)
            d1 = pltpu.async_copy(v_hbm.at[src_v.at[j + 1]], rows1, sem1)
            d0.wait()
            s0 = pltpu.async_copy(rows0, acc.at[dst_v.at[j]], sem2, add=True)
            d1.wait()
            s1 = pltpu.async_copy(rows1, acc.at[dst_v.at[j + 1]], sem3,
                                  add=True)
            s0.wait()
            s1.wait()
            return carry

        lax.fori_loop(0, NG // 2, step, 0)
        if NG % 2:
            pltpu.async_copy(v_hbm.at[src_v.at[NG - 1]], rows0, sem0).wait()
            pltpu.sync_copy(rows0, acc.at[dst_v.at[NG - 1]], add=True)

        if with_ew:
            pltpu.sync_copy(a0_hbm, a0_v)

            def ewstep(j, carry):
                for k in range(CW // 16):
                    si = src_v[j, pl.ds(k * 16, 16)]
                    di = dst_v[j, pl.ds(k * 16, 16)]
                    av = plsc.load_gather(a0_v, [si])
                    bv = plsc.load_gather(a0_v, [di])
                    ew_v[j, pl.ds(k * 16, 16)] = av * bv
                return carry

            lax.fori_loop(0, NG, ewstep, 0)
            pltpu.sync_copy(ew_v, ew_hbm.at[b])

        plsc.subcore_barrier()
        pltpu.sync_copy(acc.at[pl.ds(s * RPT, RPT)],
                        out_hbm.at[c, pl.ds(s * RPT, RPT)])

    return pl.kernel(body,
                     out_type=tuple(out_type) if with_ew else out_type[0],
                     mesh=mesh, scratch_types=scratch,
                     compiler_params=pltpu.CompilerParams(
                         use_tc_tiling_on_sc=False,
                         needs_layout_passes=False))


def _pre_body(x_ref, w_ref, o_ref):
    o_ref[...] = _dot(x_ref[...], w_ref[...])


def _gin_body(p0, p1, z, w1, b1, w2, b2, o_ref):
    h = p0[...] + p1[...] + z[...]
    h1 = _relu(_dot(h, w1[...]) + b1[...])
    o_ref[...] = _relu(_dot(h1, w2[...]) + b2[...])


def _gin0_body(pa0, pa1, pb0, pb1, z, w1, b1, w2, b2, o_ref):
    agg = jnp.concatenate([pa0[...] + pa1[...], pb0[...] + pb1[...]], 1)
    h = agg + z[...]
    h1 = _relu(_dot(h, w1[...]) + b1[...])
    o_ref[...] = _relu(_dot(h1, w2[...]) + b2[...])


def _big_body(p0, p1, z, ucat, w1, b1, w2, b2, wi1, bi1, wi2, bi2, wi3, bi3,
              z1_o, as_o, vc_o):
    h = p0[...] + p1[...] + z[...]
    h1 = _relu(_dot(h, w1[...]) + b1[...])
    z1 = _relu(_dot(h1, w2[...]) + b2[...])
    z1_o[...] = z1
    mask = lax.broadcasted_iota(jnp.int32, (NP, 1), 0) < N

    def inorm(t):
        m = jnp.sum(jnp.where(mask, t, 0.0), 0, keepdims=True) / N
        d = t - m
        v = jnp.sum(jnp.where(mask, d * d, 0.0), 0, keepdims=True) / N
        return d / jnp.sqrt(v + EPS)

    h = _relu(inorm(_dot(z1, wi1[...]) + bi1[...]))
    h = _relu(inorm(_dot(h, wi2[...]) + bi2[...]))
    ib = _dot(h, wi3[...]) + bi3[...]
    e = jnp.exp(ib - jnp.max(ib, 1, keepdims=True))
    a = e / jnp.sum(e, 1, keepdims=True)
    as_o[...] = a
    u2 = ucat[...]
    vc_o[...] = jnp.concatenate(
        [u2[:, :H] * a[:, 0:1], u2[:, H:] * a[:, 1:2]], 1)


def _mid2_body(p0, p1, u, a, b1c, b1o, w2c, b2c, w2o, b2o, w1c, w1o,
               u_o, v_o):
    a0 = a[:, 0:1]
    a1 = a[:, 1:2]
    s = p0[...] + p1[...]
    hc = _relu(s[:, :H] * a0 + u[:, :H] + b1c[...])
    ho = _relu(s[:, H:] * a1 + u[:, H:] + b1o[...])
    zc = _relu(_dot(hc, w2c[...]) + b2c[...])
    zo = _relu(_dot(ho, w2o[...]) + b2o[...])
    uc = _dot(zc, w1c[...])
    uo = _dot(zo, w1o[...])
    u_o[...] = jnp.concatenate([uc, uo], 1)
    v_o[...] = jnp.concatenate([uc * a0, uo * a1], 1)


def _fin_body(p0, p1, u, a, b1c, b1o, w2c, b2c, w2o, b2o, z1, bb, perm,
              w11, b11, w12, b12, wo1, bo1, wo2, bo2, wc1, bc1, wc2, bc2,
              wco1, bco1, wco2, bco2, ws1, bs1, ws2, bs2,
              h1_o, g1_o, hm_o, gm_o, hco_o, hres_o, hs_o):
    a0 = a[:, 0:1]
    a1 = a[:, 1:2]
    s = p0[...] + p1[...]
    hc = _relu(s[:, :H] * a0 + u[:, :H] + b1c[...])
    ho = _relu(s[:, H:] * a1 + u[:, H:] + b1o[...])
    z_m = _relu(_dot(hc, w2c[...]) + b2c[...])
    z_r = _relu(_dot(ho, w2o[...]) + b2o[...])
    oh = (bb[...] == lax.broadcasted_iota(jnp.int32, (G, 1), 0)
          ).astype(jnp.float32)
    g1 = _dot(oh, z1[...])
    gm = _dot(oh, z_m)
    gr = _dot(oh, z_r)
    g1_o[...] = g1
    gm_o[...] = gm
    grp = _dot(perm[...], gr)
    gmp = _dot(perm[...], gm)

    def fc(t, w1_, b1_, w2_, b2_):
        return _dot(_relu(_dot(t, w1_[...]) + b1_[...]), w2_[...]) + b2_[...]

    h1_o[...] = fc(g1, w11, b11, w12, b12)
    hm_o[...] = fc(gm, wo1, bo1, wo2, bo2)
    hco_o[...] = fc(jnp.concatenate([grp, gm], 1), wco1, bco1, wco2, bco2)
    hres_o[...] = fc(gr, wc1, bc1, wc2, bc2)
    hs_o[...] = fc(jnp.concatenate([gr, gmp], 1), ws1, bs1, ws2, bs2)


def _tc(body, out_shape):
    return pl.pallas_call(body, out_shape=out_shape)


def kernel(x, edge_index, batch, y, params):
    f32 = jnp.float32
    sds = jax.ShapeDtypeStruct

    x_pad = jnp.pad(x, ((0, NP - N), (0, 0)))

    def pad_idx(v):
        vb = v.reshape(NW, EPT)
        fill = N + (jnp.arange(EPT_PAD - EPT, dtype=jnp.int32) % (NP - N))
        fill = jnp.broadcast_to(fill, (NW, EPT_PAD - EPT))
        vb = jnp.concatenate([vb, fill], 1)
        return vb.reshape(NW, NG, CW)

    srcs = pad_idx(edge_index[0])
    dsts = pad_idx(edge_index[1])
    bb = jnp.pad(batch, (0, NP - N), constant_values=G).reshape(1, NP)
    zeros32 = jnp.zeros((NP, H), f32)
    zeros64 = jnp.zeros((NP, 2 * H), f32)
    ridx = jax.random.permutation(jax.random.key(42), G)
    perm = jnp.zeros((G, G), f32).at[jnp.arange(G), ridx].set(1.0)

    def r2(b):
        return b.reshape(1, -1)

    sc32 = _make_sc_scatter(H, False)
    sc64ew = _make_sc_scatter(2 * H, True)
    sc64 = _make_sc_scatter(2 * H, False)

    gp = params["gcn1"]
    cp = params["context"]
    op = params["objects"]

    w1cat0 = jnp.concatenate([cp[0][0], op[0][0]], 1)
    ucat = _tc(_pre_body, sds((NP, 2 * H), f32))(x_pad, w1cat0)

    pa = sc64(x_pad[:, :2 * H], srcs, dsts, zeros64)
    pb = sc64(x_pad[:, 2 * H:], srcs, dsts, zeros64)
    z = _tc(_gin0_body, sds((NP, H), f32))(
        pa[0], pa[1], pb[0], pb[1], x_pad,
        gp[0][0], r2(gp[0][1]), gp[0][2], r2(gp[0][3]))
    p = sc32(z, srcs, dsts, zeros32)
    z = _tc(_gin_body, sds((NP, H), f32))(
        p[0], p[1], z, gp[1][0], r2(gp[1][1]), gp[1][2], r2(gp[1][3]))
    p = sc32(z, srcs, dsts, zeros32)

    wib = params["mlp_IB"]
    z1, assign, vcat = _tc(
        _big_body,
        (sds((NP, H), f32), sds((NP, 2), f32), sds((NP, 2 * H), f32)))(
        p[0], p[1], z, ucat, gp[2][0], r2(gp[2][1]), gp[2][2], r2(gp[2][3]),
        wib[0], r2(wib[1]), wib[2], r2(wib[3]), wib[4], r2(wib[5]))

    a0flat = assign[:, 0]
    p, ew = sc64ew(vcat, srcs, dsts, zeros64, a0flat)
    for l in range(2):
        args = (p[0], p[1], ucat, assign,
                r2(cp[l][1]), r2(op[l][1]), cp[l][2], r2(cp[l][3]),
                op[l][2], r2(op[l][3]), cp[l + 1][0], op[l + 1][0])
        ucat, vcat = _tc(
            _mid2_body, (sds((NP, 2 * H), f32), sds((NP, 2 * H), f32)))(*args)
        p = sc64(vcat, srcs, dsts, zeros64)

    m1 = params["mlp1"]
    mo = params["mlp_o"]
    mc = params["mlp_c"]
    mco = params["mlp_co"]
    ms = params["mlp_co_s"]
    h1, g1, hm, gm, hco, hres, hs = _tc(
        _fin_body,
        (sds((G, C), f32), sds((G, H), f32), sds((G, C), f32),
         sds((G, H), f32), sds((G, C), f32), sds((G, C), f32),
         sds((G, C), f32)))(
        p[0], p[1], ucat, assign,
        r2(cp[2][1]), r2(op[2][1]), cp[2][2], r2(cp[2][3]),
        op[2][2], r2(op[2][3]), z1, bb, perm,
        m1[0], r2(m1[1]), m1[2], r2(m1[3]),
        mo[0], r2(mo[1]), mo[2], r2(mo[3]),
        mc[0], r2(mc[1]), mc[2], r2(mc[3]),
        mco[0], r2(mco[1]), mco[2], r2(mco[3]),
        ms[0], r2(ms[1]), ms[2], r2(ms[3]))

    assignment = assign[:N]
    a0_out = assignment[:, 0]
    ew_o = ew.reshape(NW, EPT_PAD)[:, :EPT].reshape(E, 1)
    y_shuf = y[ridx]
    return (h1, g1, hm, gm, assignment, hco, hres, ew_o, a0_out, hs, y_shuf)
